# baseline (speedup 1.0000x reference)
import jax
import jax.numpy as jnp
from jax import lax
from jax.experimental import pallas as pl
from jax.experimental.pallas import tpu as pltpu
from jax.experimental.pallas import tpu_sc as plsc
import functools

B = 4096
L = 200
C = 128
NW = 32
ROWS = B // NW
RW = C * L
LCHUNKS = (L + 15) // 16

_mesh = plsc.VectorSubcoreMesh(
    core_axis_name="c", subcore_axis_name="s", num_cores=2, num_subcores=16
)


@functools.partial(
    pl.kernel,
    out_type=jax.ShapeDtypeStruct((B, L, C), jnp.float32),
    mesh=_mesh,
    compiler_params=pltpu.CompilerParams(needs_layout_passes=False),
    scratch_types=[
        pltpu.VMEM((L, ROWS), jnp.int32),
        pltpu.VMEM((L, C), jnp.float32),
        pltpu.VMEM((L, C), jnp.float32),
        pltpu.SemaphoreType.DMA,
        pltpu.SemaphoreType.DMA,
    ],
)
def _onehot_sc(xt_hbm, out_hbm, xref, buf0, buf1, sem0, sem1):
    cid = lax.axis_index("c")
    sid = lax.axis_index("s")
    wid = sid * 2 + cid
    b0 = wid * ROWS

    pltpu.sync_copy(xt_hbm.at[:, pl.ds(b0, ROWS)], xref)

    zeros16 = jnp.zeros((16,), jnp.float32)
    ones16 = jnp.ones((16,), jnp.float32)
    iota = lax.iota(jnp.int32, 16)

    def zero_fill(buf):
        @pl.loop(0, L)
        def _zero(l):
            for i in range(C // 16):
                buf[l, pl.ds(i * 16, 16)] = zeros16

    def scatter_chunk(off, r, rvec, buf, val):
        lvec = iota + off
        cvec = plsc.load_gather(xref, [lvec, rvec])
        plsc.store_scatter(buf, [lvec, cvec], val)

    def scatter_row(r, buf, val):
        rvec = iota * 0 + r

        @pl.loop(0, LCHUNKS - 1)
        def _chunks(i):
            scatter_chunk(i * 16, r, rvec, buf, val)

        scatter_chunk(L - 16, r, rvec, buf, val)

    def dma_out(buf, r, sem):
        pltpu.async_copy(buf, out_hbm.at[b0 + r], sem)

    def wait_out(buf, sem):
        pltpu.make_async_copy(buf, out_hbm.at[b0], sem).wait()

    zero_fill(buf0)
    scatter_row(0, buf0, ones16)
    dma_out(buf0, 0, sem0)
    zero_fill(buf1)
    scatter_row(1, buf1, ones16)
    dma_out(buf1, 1, sem1)

    @pl.loop(1, ROWS // 2)
    def _main(rp):
        r0 = rp * 2
        r1 = r0 + 1
        wait_out(buf0, sem0)
        scatter_row(r0 - 2, buf0, zeros16)
        scatter_row(r0, buf0, ones16)
        dma_out(buf0, r0, sem0)
        wait_out(buf1, sem1)
        scatter_row(r1 - 2, buf1, zeros16)
        scatter_row(r1, buf1, ones16)
        dma_out(buf1, r1, sem1)

    wait_out(buf0, sem0)
    wait_out(buf1, sem1)


def kernel(x):
    return jnp.swapaxes(_onehot_sc(x.T), 1, 2)


# speedup vs baseline: 1.0167x; 1.0167x over previous
import jax
import jax.numpy as jnp
from jax import lax
from jax.experimental import pallas as pl
from jax.experimental.pallas import tpu as pltpu
from jax.experimental.pallas import tpu_sc as plsc
import functools

B = 4096
L = 200
C = 128
NW = 32
ROWS = B // NW
RW = C * L
LCHUNKS = (L + 15) // 16

_mesh = plsc.VectorSubcoreMesh(
    core_axis_name="c", subcore_axis_name="s", num_cores=2, num_subcores=16
)


@functools.partial(
    pl.kernel,
    out_type=jax.ShapeDtypeStruct((B, L, C), jnp.float32),
    mesh=_mesh,
    compiler_params=pltpu.CompilerParams(needs_layout_passes=False),
    scratch_types=[
        pltpu.VMEM((L, ROWS), jnp.int32),
        pltpu.VMEM((L, C), jnp.float32),
        pltpu.VMEM((L, C), jnp.float32),
        pltpu.SemaphoreType.DMA,
        pltpu.SemaphoreType.DMA,
        pltpu.SemaphoreType.DMA,
    ],
)
def _onehot_sc(xt_hbm, out_hbm, xref, buf0, buf1, sem0, sem1, semx):
    cid = lax.axis_index("c")
    sid = lax.axis_index("s")
    wid = sid * 2 + cid
    b0 = wid * ROWS

    xload = pltpu.async_copy(xt_hbm.at[:, pl.ds(b0, ROWS)], xref, semx)

    zeros16 = jnp.zeros((16,), jnp.float32)
    ones16 = jnp.ones((16,), jnp.float32)
    iota = lax.iota(jnp.int32, 16)

    def zero_fill(buf):
        @pl.loop(0, L)
        def _zero(l):
            for i in range(C // 16):
                buf[l, pl.ds(i * 16, 16)] = zeros16

    def scatter_row(r, buf, val):
        rvec = iota * 0 + r
        for i in range(LCHUNKS):
            off = min(i * 16, L - 16)
            lvec = iota + off
            cvec = plsc.load_gather(xref, [lvec, rvec])
            plsc.store_scatter(buf, [lvec, cvec], val)

    def dma_out(buf, r, sem):
        pltpu.async_copy(buf, out_hbm.at[b0 + r], sem)

    def wait_out(buf, sem):
        pltpu.make_async_copy(buf, out_hbm.at[b0], sem).wait()

    zero_fill(buf0)
    xload.wait()
    scatter_row(0, buf0, ones16)
    dma_out(buf0, 0, sem0)
    zero_fill(buf1)
    scatter_row(1, buf1, ones16)
    dma_out(buf1, 1, sem1)

    @pl.loop(1, ROWS // 2)
    def _main(rp):
        r0 = rp * 2
        r1 = r0 + 1
        wait_out(buf0, sem0)
        scatter_row(r0 - 2, buf0, zeros16)
        scatter_row(r0, buf0, ones16)
        dma_out(buf0, r0, sem0)
        wait_out(buf1, sem1)
        scatter_row(r1 - 2, buf1, zeros16)
        scatter_row(r1, buf1, ones16)
        dma_out(buf1, r1, sem1)

    wait_out(buf0, sem0)
    wait_out(buf1, sem1)


def kernel(x):
    return jnp.swapaxes(_onehot_sc(x.T), 1, 2)
